# baseline (speedup 1.0000x reference)
import functools

import jax
import jax.numpy as jnp
from jax.experimental import pallas as pl
from jax.experimental.pallas import tpu as pltpu


def _chamfer_body(x1_ref, x2_ref, out_ref, rowacc_ref, *, nj_blocks, inv_n):
    j = pl.program_id(1)

    x1 = x1_ref[0]
    x2 = x2_ref[0]

    t = jax.lax.dot_general(
        x1 * -2.0, x2, (((1,), (1,)), ((), ())),
        preferred_element_type=jnp.float32,
    )
    sq1 = jnp.sum(x1 * x1, axis=1, keepdims=True)
    sq2 = jax.lax.dot_general(
        jnp.ones((1, 3), jnp.float32), x2 * x2, (((1,), (1,)), ((), ())),
        preferred_element_type=jnp.float32,
    )

    row_min = jnp.min(t + sq2, axis=1, keepdims=True)

    @pl.when(j == 0)
    def _init():
        rowacc_ref[...] = row_min
        out_ref[0, 0, 0] = 0.0

    @pl.when(j > 0)
    def _acc():
        rowacc_ref[...] = jnp.minimum(rowacc_ref[...], row_min)

    col_min = jnp.min(t + sq1, axis=0, keepdims=True)
    out_ref[0, 0, 0] += (jnp.sum(col_min) + jnp.sum(sq2)) * inv_n

    @pl.when(j == nj_blocks - 1)
    def _flush():
        out_ref[0, 0, 0] += (jnp.sum(rowacc_ref[...]) + jnp.sum(sq1)) * inv_n


def kernel(xyz1, xyz2):
    B, N1, _ = xyz1.shape
    _, N2, _ = xyz2.shape
    BJ = 2048
    nj_blocks = N2 // BJ

    body = functools.partial(
        _chamfer_body, nj_blocks=nj_blocks, inv_n=1.0 / float(B * N1)
    )

    partial = pl.pallas_call(
        body,
        grid=(B, nj_blocks),
        in_specs=[
            pl.BlockSpec((1, N1, 3), lambda b, j: (b, 0, 0)),
            pl.BlockSpec((1, BJ, 3), lambda b, j: (b, j, 0)),
        ],
        out_specs=pl.BlockSpec(
            (1, 1, 1), lambda b, j: (b, 0, 0), memory_space=pltpu.SMEM
        ),
        out_shape=jax.ShapeDtypeStruct((B, 1, 1), jnp.float32),
        scratch_shapes=[pltpu.VMEM((N1, 1), jnp.float32)],
        compiler_params=pltpu.CompilerParams(
            dimension_semantics=("parallel", "arbitrary"),
        ),
    )(xyz1, xyz2)
    return jnp.sum(partial)


# speedup vs baseline: 2.2885x; 2.2885x over previous
import functools

import jax
import jax.numpy as jnp
from jax.experimental import pallas as pl
from jax.experimental.pallas import tpu as pltpu


def _chamfer_body(x1_ref, x2_ref, out_ref, *, inv_n):
    x1 = x1_ref[0]
    x2 = x2_ref[0]

    t = jax.lax.dot_general(
        x1 * -2.0, x2, (((0,), (0,)), ((), ())),
        preferred_element_type=jnp.float32,
    )
    sq1 = jnp.sum(x1 * x1, axis=0, keepdims=True)
    sq2 = jnp.sum(x2 * x2, axis=0, keepdims=True)

    row_min = jnp.min(t + sq2, axis=1, keepdims=True)
    col_min = jnp.min(t + sq1.T, axis=0, keepdims=True)

    total = jnp.sum(row_min) + jnp.sum(col_min) + jnp.sum(sq1) + jnp.sum(sq2)
    out_ref[0, 0, 0] = total * inv_n


def kernel(xyz1, xyz2):
    B, N1, _ = xyz1.shape
    _, N2, _ = xyz2.shape

    x1t = jnp.transpose(xyz1, (0, 2, 1))
    x2t = jnp.transpose(xyz2, (0, 2, 1))

    body = functools.partial(_chamfer_body, inv_n=1.0 / float(B * N1))

    partial = pl.pallas_call(
        body,
        grid=(B,),
        in_specs=[
            pl.BlockSpec((1, 3, N1), lambda b: (b, 0, 0)),
            pl.BlockSpec((1, 3, N2), lambda b: (b, 0, 0)),
        ],
        out_specs=pl.BlockSpec(
            (1, 1, 1), lambda b: (b, 0, 0), memory_space=pltpu.SMEM
        ),
        out_shape=jax.ShapeDtypeStruct((B, 1, 1), jnp.float32),
        compiler_params=pltpu.CompilerParams(
            dimension_semantics=("parallel",),
        ),
    )(x1t, x2t)
    return jnp.sum(partial)


# speedup vs baseline: 2.3185x; 1.0131x over previous
import functools

import jax
import jax.numpy as jnp
from jax.experimental import pallas as pl
from jax.experimental.pallas import tpu as pltpu


def _chamfer_body(x1_ref, x2_ref, out_ref, *, inv_n):
    b = pl.program_id(0)
    x1 = x1_ref[0]
    x2 = x2_ref[0]

    t = jax.lax.dot_general(
        x1 * -2.0, x2, (((0,), (0,)), ((), ())),
        preferred_element_type=jnp.float32,
    )
    sq1 = jnp.sum(x1 * x1, axis=0, keepdims=True)
    sq2 = jnp.sum(x2 * x2, axis=0, keepdims=True)

    row_min = jnp.min(t + sq2, axis=1, keepdims=True)
    col_min = jnp.min(t + sq1.T, axis=0, keepdims=True)

    total = jnp.sum(row_min) + jnp.sum(col_min) + jnp.sum(sq1) + jnp.sum(sq2)

    @pl.when(b == 0)
    def _zero():
        out_ref[0, 0, 0] = 0.0

    out_ref[0, 0, 0] += total * inv_n


def kernel(xyz1, xyz2):
    B, N1, _ = xyz1.shape
    _, N2, _ = xyz2.shape

    x1t = jnp.transpose(xyz1, (0, 2, 1))
    x2t = jnp.transpose(xyz2, (0, 2, 1))

    body = functools.partial(_chamfer_body, inv_n=1.0 / float(B * N1))

    partial = pl.pallas_call(
        body,
        grid=(B,),
        in_specs=[
            pl.BlockSpec((1, 3, N1), lambda b: (b, 0, 0)),
            pl.BlockSpec((1, 3, N2), lambda b: (b, 0, 0)),
        ],
        out_specs=pl.BlockSpec(
            (1, 1, 1), lambda b: (0, 0, 0), memory_space=pltpu.SMEM
        ),
        out_shape=jax.ShapeDtypeStruct((1, 1, 1), jnp.float32),
        compiler_params=pltpu.CompilerParams(
            dimension_semantics=("arbitrary",),
        ),
    )(x1t, x2t)
    return partial[0, 0, 0]
